# Initial kernel scaffold; baseline (speedup 1.0000x reference)
import functools

import jax
import jax.numpy as jnp
from jax import lax
from jax.experimental import pallas as pl
from jax.experimental.pallas import tpu as pltpu
from jax.experimental.pallas import tpu_sc as plsc

N = 10000
E = 640000
NSUB = 16
NCORE = 2
CHUNK = 112
EB = 80


def _deg_body(dst_hbm, ew_hbm, zero1_hbm, out_hbm, dstv, ewv, deg_sh):
    c = lax.axis_index("c")
    s = lax.axis_index("s")
    @pl.when(s == 0)
    def _():
        pltpu.sync_copy(zero1_hbm, deg_sh)
    plsc.subcore_barrier()

    per_tile = E // (NCORE * NSUB)
    base0 = (c * NSUB + s) * per_tile

    def blk(i, carry):
        base = base0 + i * EB
        pltpu.sync_copy(dst_hbm.at[pl.ds(base, EB)], dstv)
        pltpu.sync_copy(ew_hbm.at[pl.ds(base, EB)], ewv)
        pltpu.sync_copy(ewv, deg_sh.at[dstv], add=True)
        return carry

    lax.fori_loop(0, per_tile // EB, blk, 0)
    plsc.subcore_barrier()
    @pl.when(s == 0)
    def _():
        pltpu.sync_copy(deg_sh, out_hbm.at[c])


def _sc_deg(dst, ew, zero1):
    f = pl.kernel(
        _deg_body,
        out_type=jax.ShapeDtypeStruct((NCORE, N), jnp.float32),
        mesh=plsc.VectorSubcoreMesh(core_axis_name="c", subcore_axis_name="s"),
        scratch_types=[
            pltpu.VMEM((EB,), jnp.int32),
            pltpu.VMEM((EB,), jnp.float32),
            pltpu.VMEM_SHARED((N,), jnp.float32),
        ],
    )
    return f(dst, ew, zero1)


def _main_body(xw_hbm, src_hbm, dst_hbm, ew_hbm, dinv_hbm, zero2_hbm, out_hbm,
               srcv, srcadj, dstv, ewv, scalev, rows_v, dinv_v, acc_sh, sem):
    c = lax.axis_index("c")
    s = lax.axis_index("s")
    pltpu.sync_copy(dinv_hbm, dinv_v)
    @pl.when(s == 0)
    def _():
        pltpu.sync_copy(zero2_hbm, acc_sh)
    plsc.subcore_barrier()

    per_tile = E // NSUB
    base0 = s * per_tile
    row_off = c * N

    def blk(i, carry):
        base = base0 + i * EB
        pltpu.sync_copy(src_hbm.at[pl.ds(base, EB)], srcv)
        pltpu.sync_copy(dst_hbm.at[pl.ds(base, EB)], dstv)
        pltpu.sync_copy(ew_hbm.at[pl.ds(base, EB)], ewv)
        for g in range(EB // 16):
            sidx = srcv[pl.ds(g * 16, 16)]
            dv = plsc.load_gather(dinv_v, [sidx])
            scalev[pl.ds(g * 16, 16)] = ewv[pl.ds(g * 16, 16)] * dv
            srcadj[pl.ds(g * 16, 16)] = sidx + row_off
        pltpu.async_copy(xw_hbm.at[srcadj], rows_v, sem).wait()

        def erow(e, carry2):
            sb = plsc.load_gather(scalev, [jnp.full((16,), e, jnp.int32)])
            for f in range(CHUNK // 16):
                rows_v[e, pl.ds(f * 16, 16)] = rows_v[e, pl.ds(f * 16, 16)] * sb
            return carry2

        lax.fori_loop(0, EB, erow, 0)
        pltpu.sync_copy(rows_v, acc_sh.at[dstv], add=True)
        return carry

    lax.fori_loop(0, per_tile // EB, blk, 0)
    plsc.subcore_barrier()
    @pl.when(s == 0)
    def _():
        pltpu.sync_copy(acc_sh, out_hbm.at[pl.ds(row_off, N)])


def _sc_main(xw_cat, src, dst, ew, dinv, zero2):
    f = pl.kernel(
        _main_body,
        out_type=jax.ShapeDtypeStruct((NCORE * N, CHUNK), jnp.float32),
        mesh=plsc.VectorSubcoreMesh(core_axis_name="c", subcore_axis_name="s"),
        scratch_types=[
            pltpu.VMEM((EB,), jnp.int32),
            pltpu.VMEM((EB,), jnp.int32),
            pltpu.VMEM((EB,), jnp.int32),
            pltpu.VMEM((EB,), jnp.float32),
            pltpu.VMEM((EB,), jnp.float32),
            pltpu.VMEM((EB, CHUNK), jnp.float32),
            pltpu.VMEM((N,), jnp.float32),
            pltpu.VMEM_SHARED((N, CHUNK), jnp.float32),
            pltpu.SemaphoreType.DMA,
        ],
    )
    return f(xw_cat, src, dst, ew, dinv, zero2)


def _front_body(xT_ref, w1_ref, b1_ref, wp_ref, out_ref):
    nb = xT_ref.shape[1]
    cols = []
    for t in range(13):
        y = jnp.dot(xT_ref[t], w1_ref[0], preferred_element_type=jnp.float32)
        for k in range(1, 8):
            y = y + jnp.dot(xT_ref[t + k], w1_ref[k],
                            preferred_element_type=jnp.float32)
        y = y + b1_ref[:]
        h = y[:, :64] * jax.nn.sigmoid(y[:, 64:])
        cols.append(jnp.dot(h, wp_ref[:], preferred_element_type=jnp.float32))
    c0 = jnp.concatenate(cols[0:7], axis=1)
    c1 = jnp.concatenate(cols[7:13] + [jnp.zeros((nb, 16), jnp.float32)],
                         axis=1)
    out_ref[0] = c0
    out_ref[1] = c1


def _tc_front(xT, w1k, b1r, wp, nb=1000):
    grid = (N // nb,)
    return pl.pallas_call(
        _front_body,
        grid=grid,
        in_specs=[
            pl.BlockSpec((20, nb, 128), lambda i: (0, i, 0)),
            pl.BlockSpec((8, 128, 128), lambda i: (0, 0, 0)),
            pl.BlockSpec((1, 128), lambda i: (0, 0)),
            pl.BlockSpec((64, 16), lambda i: (0, 0)),
        ],
        out_specs=pl.BlockSpec((2, nb, CHUNK), lambda i: (0, i, 0)),
        out_shape=jax.ShapeDtypeStruct((2, N, CHUNK), jnp.float32),
    )(xT, w1k, b1r, wp)


def _dinv_body(deg_ref, out_ref):
    d = deg_ref[0] + deg_ref[1] + 1.0
    out_ref[0] = jnp.where(d > 0, lax.rsqrt(d), 0.0)


def _tc_dinv(deg_p):
    return pl.pallas_call(
        _dinv_body,
        out_shape=jax.ShapeDtypeStruct((1, N), jnp.float32),
    )(deg_p)


def _epi_body(acc_ref, xw_ref, dinv_ref, bsgc_ref, w2_ref, b2_ref, out_ref):
    a = jnp.concatenate([acc_ref[0], acc_ref[1][:, :96]], axis=1)
    xwf = jnp.concatenate([xw_ref[0], xw_ref[1][:, :96]], axis=1)
    dv = dinv_ref[0][:, None]
    z = dv * a + (dv * dv) * xwf + bsgc_ref[:]
    z = jnp.maximum(z, 0.0)
    for u in range(6):
        y = jnp.dot(z[:, u * 16:u * 16 + 128], w2_ref[:],
                    preferred_element_type=jnp.float32) + b2_ref[:]
        out_ref[u] = y[:, :64] * jax.nn.sigmoid(y[:, 64:])


def _tc_epi(acc, xw, dinv2d, bsgc, w2f, b2r, nb=1000):
    grid = (N // nb,)
    return pl.pallas_call(
        _epi_body,
        grid=grid,
        in_specs=[
            pl.BlockSpec((2, nb, CHUNK), lambda i: (0, i, 0)),
            pl.BlockSpec((2, nb, CHUNK), lambda i: (0, i, 0)),
            pl.BlockSpec((1, nb), lambda i: (0, i)),
            pl.BlockSpec((1, 208), lambda i: (0, 0)),
            pl.BlockSpec((128, 128), lambda i: (0, 0)),
            pl.BlockSpec((1, 128), lambda i: (0, 0)),
        ],
        out_specs=pl.BlockSpec((6, nb, 64), lambda i: (0, i, 0)),
        out_shape=jax.ShapeDtypeStruct((6, N, 64), jnp.float32),
    )(acc, xw, dinv2d, bsgc, w2f, b2r)


def kernel(x, edge_index, edge_attr, batch,
           w_tgc1, b_tgc1, w_sgc, b_sgc, w_tgc2, b_tgc2):
    src = edge_index[0]
    dst = edge_index[1]

    xT = jnp.transpose(x, (2, 0, 1))
    w1k = jnp.transpose(w_tgc1, (2, 1, 0))
    b1r = b_tgc1.reshape(1, 128)
    w2f = jnp.transpose(w_tgc2, (2, 1, 0)).reshape(128, 128)
    b2r = b_tgc2.reshape(1, 128)
    bsgc = jnp.tile(b_sgc, 13).reshape(1, 208)
    zero1 = jnp.zeros((N,), jnp.float32)
    zero2 = jnp.zeros((N, CHUNK), jnp.float32)

    deg_p = _sc_deg(dst, edge_attr, zero1)
    dinv2d = _tc_dinv(deg_p)
    dinv = dinv2d.reshape(N)

    xw = _tc_front(xT, w1k, b1r, w_sgc)
    xw_cat = xw.reshape(NCORE * N, CHUNK)

    acc_cat = _sc_main(xw_cat, src, dst, edge_attr, dinv, zero2)
    acc = acc_cat.reshape(2, N, CHUNK)

    res = _tc_epi(acc, xw, dinv2d, bsgc, w2f, b2r)
    return jnp.transpose(res, (1, 2, 0))


# speedup vs baseline: 45.3786x; 45.3786x over previous
import functools

import jax
import jax.numpy as jnp
from jax import lax
from jax.experimental import pallas as pl
from jax.experimental.pallas import tpu as pltpu
from jax.experimental.pallas import tpu_sc as plsc

N = 10000
E = 640000
NSUB = 16
NCORE = 2
CHUNK = 128
REAL = 104
EB = 80


def _deg_body(dst_hbm, ew_hbm, zero1_hbm, out_hbm, dstv, ewv, deg_sh):
    c = lax.axis_index("c")
    s = lax.axis_index("s")
    @pl.when(s == 0)
    def _():
        pltpu.sync_copy(zero1_hbm, deg_sh)
    plsc.subcore_barrier()

    per_tile = E // (NCORE * NSUB)
    base0 = (c * NSUB + s) * per_tile

    def blk(i, carry):
        base = base0 + i * EB
        pltpu.sync_copy(dst_hbm.at[pl.ds(base, EB)], dstv)
        pltpu.sync_copy(ew_hbm.at[pl.ds(base, EB)], ewv)
        pltpu.sync_copy(ewv, deg_sh.at[dstv], add=True)
        return carry

    lax.fori_loop(0, per_tile // EB, blk, 0)
    plsc.subcore_barrier()
    @pl.when(s == 0)
    def _():
        pltpu.sync_copy(deg_sh, out_hbm.at[c])


def _sc_deg(dst, ew, zero1):
    f = pl.kernel(
        _deg_body,
        out_type=jax.ShapeDtypeStruct((NCORE, N), jnp.float32),
        mesh=plsc.VectorSubcoreMesh(core_axis_name="c", subcore_axis_name="s"),
        scratch_types=[
            pltpu.VMEM((EB,), jnp.int32),
            pltpu.VMEM((EB,), jnp.float32),
            pltpu.VMEM_SHARED((N,), jnp.float32),
        ],
    )
    return f(dst, ew, zero1)


def _main_body(xw_hbm, src_hbm, dst_hbm, ew_hbm, zero2_hbm, out_hbm,
               srcadj, dstv, ewv, rows_v, acc_sh, sem):
    c = lax.axis_index("c")
    s = lax.axis_index("s")
    @pl.when(s == 0)
    def _():
        pltpu.sync_copy(zero2_hbm, acc_sh)
    plsc.subcore_barrier()

    per_tile = E // NSUB
    base0 = s * per_tile
    row_off = c * N

    def blk(i, carry):
        base = base0 + i * EB
        pltpu.sync_copy(src_hbm.at[pl.ds(base, EB)], srcadj)
        pltpu.sync_copy(dst_hbm.at[pl.ds(base, EB)], dstv)
        pltpu.sync_copy(ew_hbm.at[pl.ds(base, EB)], ewv)
        for g in range(EB // 16):
            srcadj[pl.ds(g * 16, 16)] = srcadj[pl.ds(g * 16, 16)] + row_off
        pltpu.async_copy(xw_hbm.at[srcadj], rows_v, sem).wait()

        def grp(g, carry2):
            sg = ewv[pl.ds(g * 16, 16)]
            for r in range(16):
                sb = sg[jnp.full((16,), r, jnp.int32)]
                e = g * 16 + r
                for f in range(CHUNK // 16):
                    rows_v[e, pl.ds(f * 16, 16)] = (
                        rows_v[e, pl.ds(f * 16, 16)] * sb)
            return carry2

        lax.fori_loop(0, EB // 16, grp, 0)
        pltpu.sync_copy(rows_v, acc_sh.at[dstv], add=True)
        return carry

    lax.fori_loop(0, per_tile // EB, blk, 0)
    plsc.subcore_barrier()
    @pl.when(s == 0)
    def _():
        pltpu.sync_copy(acc_sh, out_hbm.at[pl.ds(row_off, N)])


def _sc_main(xw_cat, src, dst, ew, zero2):
    f = pl.kernel(
        _main_body,
        out_type=jax.ShapeDtypeStruct((NCORE * N, CHUNK), jnp.float32),
        mesh=plsc.VectorSubcoreMesh(core_axis_name="c", subcore_axis_name="s"),
        scratch_types=[
            pltpu.VMEM((EB,), jnp.int32),
            pltpu.VMEM((EB,), jnp.int32),
            pltpu.VMEM((EB,), jnp.float32),
            pltpu.VMEM((EB, CHUNK), jnp.float32),
            pltpu.VMEM_SHARED((N, CHUNK), jnp.float32),
            pltpu.SemaphoreType.DMA,
        ],
    )
    return f(xw_cat, src, dst, ew, zero2)


def _front_body(xT_ref, w1_ref, b1_ref, wp_ref, dinv_ref, out_ref):
    nb = xT_ref.shape[1]
    cols = []
    for t in range(13):
        y = jnp.dot(xT_ref[t], w1_ref[0], preferred_element_type=jnp.float32)
        for k in range(1, 8):
            y = y + jnp.dot(xT_ref[t + k], w1_ref[k],
                            preferred_element_type=jnp.float32)
        y = y + b1_ref[:]
        h = y[:, :64] * jax.nn.sigmoid(y[:, 64:])
        cols.append(jnp.dot(h, wp_ref[:], preferred_element_type=jnp.float32))
    flat = jnp.concatenate(cols, axis=1) * dinv_ref[:]
    pad = jnp.zeros((nb, CHUNK - REAL), jnp.float32)
    out_ref[0] = jnp.concatenate([flat[:, :REAL], pad], axis=1)
    out_ref[1] = jnp.concatenate([flat[:, REAL:], pad], axis=1)


def _tc_front(xT, w1k, b1r, wp, dinv_col, nb=1000):
    grid = (N // nb,)
    return pl.pallas_call(
        _front_body,
        grid=grid,
        in_specs=[
            pl.BlockSpec((20, nb, 128), lambda i: (0, i, 0)),
            pl.BlockSpec((8, 128, 128), lambda i: (0, 0, 0)),
            pl.BlockSpec((1, 128), lambda i: (0, 0)),
            pl.BlockSpec((64, 16), lambda i: (0, 0)),
            pl.BlockSpec((nb, 1), lambda i: (i, 0)),
        ],
        out_specs=pl.BlockSpec((2, nb, CHUNK), lambda i: (0, i, 0)),
        out_shape=jax.ShapeDtypeStruct((2, N, CHUNK), jnp.float32),
    )(xT, w1k, b1r, wp, dinv_col)


def _dinv_body(deg_ref, out_ref):
    d = deg_ref[0] + deg_ref[1] + 1.0
    out_ref[0] = jnp.where(d > 0, lax.rsqrt(d), 0.0)


def _tc_dinv(deg_p):
    return pl.pallas_call(
        _dinv_body,
        out_shape=jax.ShapeDtypeStruct((1, N), jnp.float32),
    )(deg_p)


def _epi_body(acc_ref, xw_ref, dinv_ref, bsgc_ref, w2_ref, b2_ref, out_ref):
    a = jnp.concatenate([acc_ref[0][:, :REAL], acc_ref[1][:, :REAL]], axis=1)
    xwf = jnp.concatenate([xw_ref[0][:, :REAL], xw_ref[1][:, :REAL]], axis=1)
    dv = dinv_ref[:]
    z = dv * (a + xwf) + bsgc_ref[:]
    z = jnp.maximum(z, 0.0)
    for u in range(6):
        y = jnp.dot(z[:, u * 16:u * 16 + 128], w2_ref[:],
                    preferred_element_type=jnp.float32) + b2_ref[:]
        out_ref[u] = y[:, :64] * jax.nn.sigmoid(y[:, 64:])


def _tc_epi(acc, xw, dinv_col, bsgc, w2f, b2r, nb=1000):
    grid = (N // nb,)
    return pl.pallas_call(
        _epi_body,
        grid=grid,
        in_specs=[
            pl.BlockSpec((2, nb, CHUNK), lambda i: (0, i, 0)),
            pl.BlockSpec((2, nb, CHUNK), lambda i: (0, i, 0)),
            pl.BlockSpec((nb, 1), lambda i: (i, 0)),
            pl.BlockSpec((1, 208), lambda i: (0, 0)),
            pl.BlockSpec((128, 128), lambda i: (0, 0)),
            pl.BlockSpec((1, 128), lambda i: (0, 0)),
        ],
        out_specs=pl.BlockSpec((6, nb, 64), lambda i: (0, i, 0)),
        out_shape=jax.ShapeDtypeStruct((6, N, 64), jnp.float32),
    )(acc, xw, dinv_col, bsgc, w2f, b2r)


def kernel(x, edge_index, edge_attr, batch,
           w_tgc1, b_tgc1, w_sgc, b_sgc, w_tgc2, b_tgc2):
    src = edge_index[0]
    dst = edge_index[1]

    xT = jnp.transpose(x, (2, 0, 1))
    w1k = jnp.transpose(w_tgc1, (2, 1, 0))
    b1r = b_tgc1.reshape(1, 128)
    w2f = jnp.transpose(w_tgc2, (2, 1, 0)).reshape(128, 128)
    b2r = b_tgc2.reshape(1, 128)
    bsgc = jnp.tile(b_sgc, 13).reshape(1, 208)
    zero1 = jnp.zeros((N,), jnp.float32)
    zero2 = jnp.zeros((N, CHUNK), jnp.float32)

    deg_p = _sc_deg(dst, edge_attr, zero1)
    dinv2d = _tc_dinv(deg_p)
    dinv_col = dinv2d.reshape(N, 1)

    xw = _tc_front(xT, w1k, b1r, w_sgc, dinv_col)
    xw_cat = xw.reshape(NCORE * N, CHUNK)

    acc_cat = _sc_main(xw_cat, src, dst, edge_attr, zero2)
    acc = acc_cat.reshape(2, N, CHUNK)

    res = _tc_epi(acc, xw, dinv_col, bsgc, w2f, b2r)
    return jnp.transpose(res, (1, 2, 0))


# speedup vs baseline: 93.7074x; 2.0650x over previous
import functools

import jax
import jax.numpy as jnp
from jax import lax
from jax.experimental import pallas as pl
from jax.experimental.pallas import tpu as pltpu
from jax.experimental.pallas import tpu_sc as plsc

N = 10000
E = 640000
NSUB = 16
NCORE = 2
CHUNK = 128
REAL = 104
EB = 80


def _deg_body(dst_hbm, ew_hbm, zero1_hbm, out_hbm, dstv, ewv, deg_sh):
    c = lax.axis_index("c")
    s = lax.axis_index("s")
    @pl.when(s == 0)
    def _():
        pltpu.sync_copy(zero1_hbm, deg_sh)
    plsc.subcore_barrier()

    per_tile = E // (NCORE * NSUB)
    base0 = (c * NSUB + s) * per_tile

    def blk(i, carry):
        base = base0 + i * EB
        pltpu.sync_copy(dst_hbm.at[pl.ds(base, EB)], dstv)
        pltpu.sync_copy(ew_hbm.at[pl.ds(base, EB)], ewv)
        pltpu.sync_copy(ewv, deg_sh.at[dstv], add=True)
        return carry

    lax.fori_loop(0, per_tile // EB, blk, 0)
    plsc.subcore_barrier()
    @pl.when(s == 0)
    def _():
        pltpu.sync_copy(deg_sh, out_hbm.at[c])


def _sc_deg(dst, ew, zero1):
    f = pl.kernel(
        _deg_body,
        out_type=jax.ShapeDtypeStruct((NCORE, N), jnp.float32),
        mesh=plsc.VectorSubcoreMesh(core_axis_name="c", subcore_axis_name="s"),
        scratch_types=[
            pltpu.VMEM((EB,), jnp.int32),
            pltpu.VMEM((EB,), jnp.float32),
            pltpu.VMEM_SHARED((N,), jnp.float32),
        ],
    )
    return f(dst, ew, zero1)


ROWS = E // 128
RPT = ROWS // NSUB
CR = 8


def _scale_rows(rows_ref, ew_st, p, jj):
    def grp(g, carry):
        sg = ew_st[p, jj, pl.ds(g * 16, 16)]
        for r in range(16):
            sb = sg[jnp.full((16,), r, jnp.int32)]
            e = g * 16 + r
            for f in range(CHUNK // 16):
                rows_ref[e, pl.ds(f * 16, 16)] = (
                    rows_ref[e, pl.ds(f * 16, 16)] * sb)
        return carry
    lax.fori_loop(0, 8, grp, 0)


def _main_body(xw_hbm, src_hbm, dst_hbm, ew_hbm, zero2_hbm, out_hbm,
               src_st, dst_st, ew_st, rows_a, rows_b,
               acc_sh, gsem, ssem, stsem):
    c = lax.axis_index("c")
    s = lax.axis_index("s")
    @pl.when(s == 0)
    def _():
        pltpu.sync_copy(zero2_hbm, acc_sh)
    plsc.subcore_barrier()

    row_off = c * N
    r0 = s * RPT
    nchunks = jnp.where(s == NSUB - 1, (ROWS - (NSUB - 1) * RPT) // CR,
                        RPT // CR)

    def stage(k, p):
        base = r0 + k * CR
        pltpu.async_copy(src_hbm.at[pl.ds(base, CR)], src_st.at[p], stsem)
        pltpu.async_copy(dst_hbm.at[pl.ds(base, CR)], dst_st.at[p], stsem)
        pltpu.async_copy(ew_hbm.at[pl.ds(base, CR)], ew_st.at[p], stsem)

    def wait_st(p):
        pltpu.make_async_copy(src_hbm.at[pl.ds(0, CR)], src_st.at[p],
                              stsem).wait()
        pltpu.make_async_copy(dst_hbm.at[pl.ds(0, CR)], dst_st.at[p],
                              stsem).wait()
        pltpu.make_async_copy(ew_hbm.at[pl.ds(0, CR)], ew_st.at[p],
                              stsem).wait()

    stage(0, 0)

    def chunk(k, carry):
        p = lax.rem(k, 2)
        wait_st(p)
        @pl.when(k + 1 < nchunks)
        def _():
            stage(k + 1, 1 - p)
        for jj in range(CR):
            for g in range(8):
                src_st[p, jj, pl.ds(g * 16, 16)] = (
                    src_st[p, jj, pl.ds(g * 16, 16)] + row_off)
        @pl.when(k > 0)
        def _():
            pltpu.make_async_copy(rows_b, acc_sh.at[dst_st.at[p, 0]],
                                  ssem).wait()
        pltpu.async_copy(xw_hbm.at[src_st.at[p, 0]], rows_a, gsem)
        for jj in range(CR):
            cur = rows_a if jj % 2 == 0 else rows_b
            nxt = rows_b if jj % 2 == 0 else rows_a
            pltpu.make_async_copy(xw_hbm.at[src_st.at[p, jj]], cur,
                                  gsem).wait()
            if jj >= 1:
                pltpu.make_async_copy(nxt, acc_sh.at[dst_st.at[p, jj - 1]],
                                      ssem).wait()
            if jj + 1 < CR:
                pltpu.async_copy(xw_hbm.at[src_st.at[p, jj + 1]], nxt, gsem)
            _scale_rows(cur, ew_st, p, jj)
            pltpu.async_copy(cur, acc_sh.at[dst_st.at[p, jj]], ssem,
                             add=True)
        return carry

    lax.fori_loop(0, nchunks, chunk, 0)
    pltpu.make_async_copy(rows_b, acc_sh.at[dst_st.at[0, 0]], ssem).wait()
    plsc.subcore_barrier()
    @pl.when(s == 0)
    def _():
        pltpu.sync_copy(acc_sh, out_hbm.at[pl.ds(row_off, N)])


def _sc_main(xw_cat, src2d, dst2d, ew2d, zero2):
    f = pl.kernel(
        _main_body,
        out_type=jax.ShapeDtypeStruct((NCORE * N, CHUNK), jnp.float32),
        mesh=plsc.VectorSubcoreMesh(core_axis_name="c", subcore_axis_name="s"),
        scratch_types=[
            pltpu.VMEM((2, CR, 128), jnp.int32),
            pltpu.VMEM((2, CR, 128), jnp.int32),
            pltpu.VMEM((2, CR, 128), jnp.float32),
            pltpu.VMEM((128, CHUNK), jnp.float32),
            pltpu.VMEM((128, CHUNK), jnp.float32),
            pltpu.VMEM_SHARED((N, CHUNK), jnp.float32),
            pltpu.SemaphoreType.DMA,
            pltpu.SemaphoreType.DMA,
            pltpu.SemaphoreType.DMA,
        ],
    )
    return f(xw_cat, src2d, dst2d, ew2d, zero2)


def _front_body(xT_ref, w1_ref, b1_ref, wp_ref, dinv_ref, out_ref):
    nb = xT_ref.shape[1]
    cols = []
    for t in range(13):
        y = jnp.dot(xT_ref[t], w1_ref[0], preferred_element_type=jnp.float32)
        for k in range(1, 8):
            y = y + jnp.dot(xT_ref[t + k], w1_ref[k],
                            preferred_element_type=jnp.float32)
        y = y + b1_ref[:]
        h = y[:, :64] * jax.nn.sigmoid(y[:, 64:])
        cols.append(jnp.dot(h, wp_ref[:], preferred_element_type=jnp.float32))
    flat = jnp.concatenate(cols, axis=1) * dinv_ref[:]
    pad = jnp.zeros((nb, CHUNK - REAL), jnp.float32)
    out_ref[0] = jnp.concatenate([flat[:, :REAL], pad], axis=1)
    out_ref[1] = jnp.concatenate([flat[:, REAL:], pad], axis=1)


def _tc_front(xT, w1k, b1r, wp, dinv_col, nb=1000):
    grid = (N // nb,)
    return pl.pallas_call(
        _front_body,
        grid=grid,
        in_specs=[
            pl.BlockSpec((20, nb, 128), lambda i: (0, i, 0)),
            pl.BlockSpec((8, 128, 128), lambda i: (0, 0, 0)),
            pl.BlockSpec((1, 128), lambda i: (0, 0)),
            pl.BlockSpec((64, 16), lambda i: (0, 0)),
            pl.BlockSpec((nb, 1), lambda i: (i, 0)),
        ],
        out_specs=pl.BlockSpec((2, nb, CHUNK), lambda i: (0, i, 0)),
        out_shape=jax.ShapeDtypeStruct((2, N, CHUNK), jnp.float32),
    )(xT, w1k, b1r, wp, dinv_col)


def _dinv_body(deg_ref, out_ref):
    d = deg_ref[0] + deg_ref[1] + 1.0
    out_ref[0] = jnp.where(d > 0, lax.rsqrt(d), 0.0)


def _tc_dinv(deg_p):
    return pl.pallas_call(
        _dinv_body,
        out_shape=jax.ShapeDtypeStruct((1, N), jnp.float32),
    )(deg_p)


def _epi_body(acc_ref, xw_ref, dinv_ref, bsgc_ref, w2_ref, b2_ref, out_ref):
    a = jnp.concatenate([acc_ref[0][:, :REAL], acc_ref[1][:, :REAL]], axis=1)
    xwf = jnp.concatenate([xw_ref[0][:, :REAL], xw_ref[1][:, :REAL]], axis=1)
    dv = dinv_ref[:]
    z = dv * (a + xwf) + bsgc_ref[:]
    z = jnp.maximum(z, 0.0)
    for u in range(6):
        y = jnp.dot(z[:, u * 16:u * 16 + 128], w2_ref[:],
                    preferred_element_type=jnp.float32) + b2_ref[:]
        out_ref[u] = y[:, :64] * jax.nn.sigmoid(y[:, 64:])


def _tc_epi(acc, xw, dinv_col, bsgc, w2f, b2r, nb=1000):
    grid = (N // nb,)
    return pl.pallas_call(
        _epi_body,
        grid=grid,
        in_specs=[
            pl.BlockSpec((2, nb, CHUNK), lambda i: (0, i, 0)),
            pl.BlockSpec((2, nb, CHUNK), lambda i: (0, i, 0)),
            pl.BlockSpec((nb, 1), lambda i: (i, 0)),
            pl.BlockSpec((1, 208), lambda i: (0, 0)),
            pl.BlockSpec((128, 128), lambda i: (0, 0)),
            pl.BlockSpec((1, 128), lambda i: (0, 0)),
        ],
        out_specs=pl.BlockSpec((6, nb, 64), lambda i: (0, i, 0)),
        out_shape=jax.ShapeDtypeStruct((6, N, 64), jnp.float32),
    )(acc, xw, dinv_col, bsgc, w2f, b2r)


def kernel(x, edge_index, edge_attr, batch,
           w_tgc1, b_tgc1, w_sgc, b_sgc, w_tgc2, b_tgc2):
    src = edge_index[0]
    dst = edge_index[1]

    xT = jnp.transpose(x, (2, 0, 1))
    w1k = jnp.transpose(w_tgc1, (2, 1, 0))
    b1r = b_tgc1.reshape(1, 128)
    w2f = jnp.transpose(w_tgc2, (2, 1, 0)).reshape(128, 128)
    b2r = b_tgc2.reshape(1, 128)
    bsgc = jnp.tile(b_sgc, 13).reshape(1, 208)
    zero1 = jnp.zeros((N,), jnp.float32)
    zero2 = jnp.zeros((N, CHUNK), jnp.float32)

    deg_p = _sc_deg(dst, edge_attr, zero1)
    dinv2d = _tc_dinv(deg_p)
    dinv_col = dinv2d.reshape(N, 1)

    xw = _tc_front(xT, w1k, b1r, w_sgc, dinv_col)
    xw_cat = xw.reshape(NCORE * N, CHUNK)

    acc_cat = _sc_main(xw_cat, src.reshape(ROWS, 128), dst.reshape(ROWS, 128),
                       edge_attr.reshape(ROWS, 128), zero2)
    acc = acc_cat.reshape(2, N, CHUNK)

    res = _tc_epi(acc, xw, dinv_col, bsgc, w2f, b2r)
    return jnp.transpose(res, (1, 2, 0))


# speedup vs baseline: 119.1625x; 1.2716x over previous
import functools

import jax
import jax.numpy as jnp
from jax import lax
from jax.experimental import pallas as pl
from jax.experimental.pallas import tpu as pltpu
from jax.experimental.pallas import tpu_sc as plsc

N = 10000
E = 640000
NSUB = 16
NCORE = 2
CHUNK = 128
REAL = 104
EB = 80


CRD = 4
DRPT = (E // 128) // (NCORE * NSUB)


def _deg_body(dst_hbm, ew_hbm, zero1_hbm, out_hbm,
              dst_st, ew_st, deg_sh, dsem, stsem):
    c = lax.axis_index("c")
    s = lax.axis_index("s")
    @pl.when(s == 0)
    def _():
        pltpu.sync_copy(zero1_hbm, deg_sh)
    plsc.subcore_barrier()

    t = c * NSUB + s
    r0 = t * DRPT
    nrows = E // 128
    nch = jnp.where(t == NCORE * NSUB - 1,
                    (nrows - (NCORE * NSUB - 1) * DRPT) // CRD, DRPT // CRD)

    def stage(k, p):
        base = r0 + k * CRD
        pltpu.async_copy(dst_hbm.at[pl.ds(base, CRD)], dst_st.at[p], stsem)
        pltpu.async_copy(ew_hbm.at[pl.ds(base, CRD)], ew_st.at[p], stsem)

    def wait_st(p):
        pltpu.make_async_copy(dst_hbm.at[pl.ds(0, CRD)], dst_st.at[p],
                              stsem).wait()
        pltpu.make_async_copy(ew_hbm.at[pl.ds(0, CRD)], ew_st.at[p],
                              stsem).wait()

    stage(0, 0)

    def chunk(k, carry):
        p = lax.rem(k, 2)
        wait_st(p)
        @pl.when(k + 1 < nch)
        def _():
            stage(k + 1, 1 - p)
        for jj in range(CRD):
            pltpu.async_copy(ew_st.at[p, jj], deg_sh.at[dst_st.at[p, jj]],
                             dsem, add=True)
        for jj in range(CRD):
            pltpu.make_async_copy(ew_st.at[p, jj],
                                  deg_sh.at[dst_st.at[p, jj]], dsem).wait()
        return carry

    lax.fori_loop(0, nch, chunk, 0)
    plsc.subcore_barrier()
    @pl.when(s == 0)
    def _():
        pltpu.sync_copy(deg_sh, out_hbm.at[c])


def _sc_deg(dst2d, ew2d, zero1):
    f = pl.kernel(
        _deg_body,
        out_type=jax.ShapeDtypeStruct((NCORE, N), jnp.float32),
        mesh=plsc.VectorSubcoreMesh(core_axis_name="c", subcore_axis_name="s"),
        scratch_types=[
            pltpu.VMEM((2, CRD, 128), jnp.int32),
            pltpu.VMEM((2, CRD, 128), jnp.float32),
            pltpu.VMEM_SHARED((N,), jnp.float32),
            pltpu.SemaphoreType.DMA,
            pltpu.SemaphoreType.DMA,
        ],
    )
    return f(dst2d, ew2d, zero1)


ROWS = E // 128
RPT = ROWS // NSUB
CR = 8


def _scale_rows(rows_ref, ew_st, p, jj):
    def grp(g, carry):
        sg = ew_st[p, jj, pl.ds(g * 16, 16)]
        for r in range(16):
            sb = sg[jnp.full((16,), r, jnp.int32)]
            e = g * 16 + r
            for f in range(7):
                rows_ref[e, pl.ds(f * 16, 16)] = (
                    rows_ref[e, pl.ds(f * 16, 16)] * sb)
        return carry
    lax.fori_loop(0, 8, grp, 0)


def _main_body(xw_hbm, src_hbm, dst_hbm, ew_hbm, zero2_hbm, out_hbm,
               src_st, dst_st, ew_st, rows_a, rows_b,
               acc_sh, gsem, ssem, stsem):
    c = lax.axis_index("c")
    s = lax.axis_index("s")
    @pl.when(s == 0)
    def _():
        pltpu.sync_copy(zero2_hbm, acc_sh)
    plsc.subcore_barrier()

    row_off = c * N
    r0 = s * RPT
    nchunks = jnp.where(s == NSUB - 1, (ROWS - (NSUB - 1) * RPT) // CR,
                        RPT // CR)

    def stage(k, p):
        base = r0 + k * CR
        pltpu.async_copy(src_hbm.at[pl.ds(base, CR)], src_st.at[p], stsem)
        pltpu.async_copy(dst_hbm.at[pl.ds(base, CR)], dst_st.at[p], stsem)
        pltpu.async_copy(ew_hbm.at[pl.ds(base, CR)], ew_st.at[p], stsem)

    def wait_st(p):
        pltpu.make_async_copy(src_hbm.at[pl.ds(0, CR)], src_st.at[p],
                              stsem).wait()
        pltpu.make_async_copy(dst_hbm.at[pl.ds(0, CR)], dst_st.at[p],
                              stsem).wait()
        pltpu.make_async_copy(ew_hbm.at[pl.ds(0, CR)], ew_st.at[p],
                              stsem).wait()

    stage(0, 0)

    def chunk(k, carry):
        p = lax.rem(k, 2)
        wait_st(p)
        @pl.when(k + 1 < nchunks)
        def _():
            stage(k + 1, 1 - p)
        for jj in range(CR):
            for g in range(8):
                src_st[p, jj, pl.ds(g * 16, 16)] = (
                    src_st[p, jj, pl.ds(g * 16, 16)] + row_off)
        @pl.when(k > 0)
        def _():
            pltpu.make_async_copy(rows_b, acc_sh.at[dst_st.at[p, 0]],
                                  ssem).wait()
        pltpu.async_copy(xw_hbm.at[src_st.at[p, 0]], rows_a, gsem)
        for jj in range(CR):
            cur = rows_a if jj % 2 == 0 else rows_b
            nxt = rows_b if jj % 2 == 0 else rows_a
            pltpu.make_async_copy(xw_hbm.at[src_st.at[p, jj]], cur,
                                  gsem).wait()
            if jj >= 1:
                pltpu.make_async_copy(nxt, acc_sh.at[dst_st.at[p, jj - 1]],
                                      ssem).wait()
            if jj + 1 < CR:
                pltpu.async_copy(xw_hbm.at[src_st.at[p, jj + 1]], nxt, gsem)
            _scale_rows(cur, ew_st, p, jj)
            pltpu.async_copy(cur, acc_sh.at[dst_st.at[p, jj]], ssem,
                             add=True)
        return carry

    lax.fori_loop(0, nchunks, chunk, 0)
    pltpu.make_async_copy(rows_b, acc_sh.at[dst_st.at[0, 0]], ssem).wait()
    plsc.subcore_barrier()
    @pl.when(s == 0)
    def _():
        pltpu.sync_copy(acc_sh, out_hbm.at[pl.ds(row_off, N)])


def _sc_main(xw_cat, src2d, dst2d, ew2d, zero2):
    f = pl.kernel(
        _main_body,
        out_type=jax.ShapeDtypeStruct((NCORE * N, CHUNK), jnp.float32),
        mesh=plsc.VectorSubcoreMesh(core_axis_name="c", subcore_axis_name="s"),
        scratch_types=[
            pltpu.VMEM((2, CR, 128), jnp.int32),
            pltpu.VMEM((2, CR, 128), jnp.int32),
            pltpu.VMEM((2, CR, 128), jnp.float32),
            pltpu.VMEM((128, CHUNK), jnp.float32),
            pltpu.VMEM((128, CHUNK), jnp.float32),
            pltpu.VMEM_SHARED((N, CHUNK), jnp.float32),
            pltpu.SemaphoreType.DMA,
            pltpu.SemaphoreType.DMA,
            pltpu.SemaphoreType.DMA,
        ],
    )
    return f(xw_cat, src2d, dst2d, ew2d, zero2)


def _front_body(xT_ref, w1_ref, b1_ref, wp_ref, dinv_ref, out_ref):
    nb = xT_ref.shape[1]
    xb = [xT_ref[t].astype(jnp.bfloat16) for t in range(20)]
    wb = [w1_ref[k].astype(jnp.bfloat16) for k in range(8)]
    cols = []
    for t in range(13):
        y = jnp.dot(xb[t], wb[0], preferred_element_type=jnp.float32)
        for k in range(1, 8):
            y = y + jnp.dot(xb[t + k], wb[k],
                            preferred_element_type=jnp.float32)
        y = y + b1_ref[:]
        h = y[:, :64] * jax.nn.sigmoid(y[:, 64:])
        cols.append(jnp.dot(h, wp_ref[:], preferred_element_type=jnp.float32))
    flat = jnp.concatenate(cols, axis=1) * dinv_ref[:]
    pad = jnp.zeros((nb, CHUNK - REAL), jnp.float32)
    out_ref[0] = jnp.concatenate([flat[:, :REAL], pad], axis=1)
    out_ref[1] = jnp.concatenate([flat[:, REAL:], pad], axis=1)


def _tc_front(xT, w1k, b1r, wp, dinv_col, nb=1000):
    grid = (N // nb,)
    return pl.pallas_call(
        _front_body,
        grid=grid,
        in_specs=[
            pl.BlockSpec((20, nb, 128), lambda i: (0, i, 0)),
            pl.BlockSpec((8, 128, 128), lambda i: (0, 0, 0)),
            pl.BlockSpec((1, 128), lambda i: (0, 0)),
            pl.BlockSpec((64, 16), lambda i: (0, 0)),
            pl.BlockSpec((nb, 1), lambda i: (i, 0)),
        ],
        out_specs=pl.BlockSpec((2, nb, CHUNK), lambda i: (0, i, 0)),
        out_shape=jax.ShapeDtypeStruct((2, N, CHUNK), jnp.float32),
    )(xT, w1k, b1r, wp, dinv_col)


def _dinv_body(deg_ref, out_ref):
    d = deg_ref[0] + deg_ref[1] + 1.0
    out_ref[0] = jnp.where(d > 0, lax.rsqrt(d), 0.0)


def _tc_dinv(deg_p):
    return pl.pallas_call(
        _dinv_body,
        out_shape=jax.ShapeDtypeStruct((1, N), jnp.float32),
    )(deg_p)


def _epi_body(acc_ref, xw_ref, dinv_ref, bsgc_ref, w2_ref, b2_ref, out_ref):
    a = jnp.concatenate([acc_ref[0][:, :REAL], acc_ref[1][:, :REAL]], axis=1)
    xwf = jnp.concatenate([xw_ref[0][:, :REAL], xw_ref[1][:, :REAL]], axis=1)
    dv = dinv_ref[:]
    z = dv * (a + xwf) + bsgc_ref[:]
    z = jnp.maximum(z, 0.0)
    for u in range(6):
        y = jnp.dot(z[:, u * 16:u * 16 + 128], w2_ref[:],
                    preferred_element_type=jnp.float32) + b2_ref[:]
        out_ref[u] = y[:, :64] * jax.nn.sigmoid(y[:, 64:])


def _tc_epi(acc, xw, dinv_col, bsgc, w2f, b2r, nb=1000):
    grid = (N // nb,)
    return pl.pallas_call(
        _epi_body,
        grid=grid,
        in_specs=[
            pl.BlockSpec((2, nb, CHUNK), lambda i: (0, i, 0)),
            pl.BlockSpec((2, nb, CHUNK), lambda i: (0, i, 0)),
            pl.BlockSpec((nb, 1), lambda i: (i, 0)),
            pl.BlockSpec((1, 208), lambda i: (0, 0)),
            pl.BlockSpec((128, 128), lambda i: (0, 0)),
            pl.BlockSpec((1, 128), lambda i: (0, 0)),
        ],
        out_specs=pl.BlockSpec((6, nb, 64), lambda i: (0, i, 0)),
        out_shape=jax.ShapeDtypeStruct((6, N, 64), jnp.float32),
    )(acc, xw, dinv_col, bsgc, w2f, b2r)


def kernel(x, edge_index, edge_attr, batch,
           w_tgc1, b_tgc1, w_sgc, b_sgc, w_tgc2, b_tgc2):
    src = edge_index[0]
    dst = edge_index[1]

    xT = jnp.transpose(x, (2, 0, 1))
    w1k = jnp.transpose(w_tgc1, (2, 1, 0))
    b1r = b_tgc1.reshape(1, 128)
    w2f = jnp.transpose(w_tgc2, (2, 1, 0)).reshape(128, 128)
    b2r = b_tgc2.reshape(1, 128)
    bsgc = jnp.tile(b_sgc, 13).reshape(1, 208)
    zero1 = jnp.zeros((N,), jnp.float32)
    zero2 = jnp.zeros((N, CHUNK), jnp.float32)

    dst2d = dst.reshape(ROWS, 128)
    ew2d = edge_attr.reshape(ROWS, 128)
    deg_p = _sc_deg(dst2d, ew2d, zero1)
    dinv2d = _tc_dinv(deg_p)
    dinv_col = dinv2d.reshape(N, 1)

    xw = _tc_front(xT, w1k, b1r, w_sgc, dinv_col)
    xw_cat = xw.reshape(NCORE * N, CHUNK)

    acc_cat = _sc_main(xw_cat, src.reshape(ROWS, 128), dst2d, ew2d, zero2)
    acc = acc_cat.reshape(2, N, CHUNK)

    res = _tc_epi(acc, xw, dinv_col, bsgc, w2f, b2r)
    return jnp.transpose(res, (1, 2, 0))


# speedup vs baseline: 121.0695x; 1.0160x over previous
import functools

import jax
import jax.numpy as jnp
from jax import lax
from jax.experimental import pallas as pl
from jax.experimental.pallas import tpu as pltpu
from jax.experimental.pallas import tpu_sc as plsc

N = 10000
E = 640000
NSUB = 16
NCORE = 2
CHUNK = 128
REAL = 104
EB = 80


CRD = 4
DRPT = (E // 128) // (NCORE * NSUB)


def _deg_body(dst_hbm, ew_hbm, zero1_hbm, out_hbm,
              dst_st, ew_st, deg_sh, dsem, stsem):
    c = lax.axis_index("c")
    s = lax.axis_index("s")
    @pl.when(s == 0)
    def _():
        pltpu.sync_copy(zero1_hbm, deg_sh)
    plsc.subcore_barrier()

    t = c * NSUB + s
    r0 = t * DRPT
    nrows = E // 128
    nch = jnp.where(t == NCORE * NSUB - 1,
                    (nrows - (NCORE * NSUB - 1) * DRPT) // CRD, DRPT // CRD)

    def stage(k, p):
        base = r0 + k * CRD
        pltpu.async_copy(dst_hbm.at[pl.ds(base, CRD)], dst_st.at[p], stsem)
        pltpu.async_copy(ew_hbm.at[pl.ds(base, CRD)], ew_st.at[p], stsem)

    def wait_st(p):
        pltpu.make_async_copy(dst_hbm.at[pl.ds(0, CRD)], dst_st.at[p],
                              stsem).wait()
        pltpu.make_async_copy(ew_hbm.at[pl.ds(0, CRD)], ew_st.at[p],
                              stsem).wait()

    stage(0, 0)

    def chunk(k, carry):
        p = lax.rem(k, 2)
        wait_st(p)
        @pl.when(k + 1 < nch)
        def _():
            stage(k + 1, 1 - p)
        for jj in range(CRD):
            pltpu.async_copy(ew_st.at[p, jj], deg_sh.at[dst_st.at[p, jj]],
                             dsem, add=True)
        for jj in range(CRD):
            pltpu.make_async_copy(ew_st.at[p, jj],
                                  deg_sh.at[dst_st.at[p, jj]], dsem).wait()
        return carry

    lax.fori_loop(0, nch, chunk, 0)
    plsc.subcore_barrier()
    @pl.when(s == 0)
    def _():
        pltpu.sync_copy(deg_sh, out_hbm.at[c])


def _sc_deg(dst2d, ew2d, zero1):
    f = pl.kernel(
        _deg_body,
        out_type=jax.ShapeDtypeStruct((NCORE, N), jnp.float32),
        mesh=plsc.VectorSubcoreMesh(core_axis_name="c", subcore_axis_name="s"),
        scratch_types=[
            pltpu.VMEM((2, CRD, 128), jnp.int32),
            pltpu.VMEM((2, CRD, 128), jnp.float32),
            pltpu.VMEM_SHARED((N,), jnp.float32),
            pltpu.SemaphoreType.DMA,
            pltpu.SemaphoreType.DMA,
        ],
    )
    return f(dst2d, ew2d, zero1)


ROWS = E // 128
RPT = ROWS // NSUB
CR = 8


def _scale_rows(rows_ref, ew_st, p, jj):
    def grp(g, carry):
        sg = ew_st[p, jj, pl.ds(g * 16, 16)]
        for r in range(16):
            sb = sg[jnp.full((16,), r, jnp.int32)]
            e = g * 16 + r
            for f in range(7):
                rows_ref[e, pl.ds(f * 16, 16)] = (
                    rows_ref[e, pl.ds(f * 16, 16)] * sb)
        return carry
    lax.fori_loop(0, 8, grp, 0)


def _main_body(xw_hbm, src_hbm, dst_hbm, ew_hbm, zero2_hbm, out_hbm,
               src_st, dst_st, ew_st, rows_a, rows_b,
               acc_sh, gsem, ssem, stsem):
    c = lax.axis_index("c")
    s = lax.axis_index("s")
    @pl.when(s == 0)
    def _():
        pltpu.sync_copy(zero2_hbm, acc_sh)
    plsc.subcore_barrier()

    row_off = c * N
    r0 = s * RPT
    nchunks = jnp.where(s == NSUB - 1, (ROWS - (NSUB - 1) * RPT) // CR,
                        RPT // CR)

    def stage(k, p):
        base = r0 + k * CR
        pltpu.async_copy(src_hbm.at[pl.ds(base, CR)], src_st.at[p], stsem)
        pltpu.async_copy(dst_hbm.at[pl.ds(base, CR)], dst_st.at[p], stsem)
        pltpu.async_copy(ew_hbm.at[pl.ds(base, CR)], ew_st.at[p], stsem)

    def wait_st(p):
        pltpu.make_async_copy(src_hbm.at[pl.ds(0, CR)], src_st.at[p],
                              stsem).wait()
        pltpu.make_async_copy(dst_hbm.at[pl.ds(0, CR)], dst_st.at[p],
                              stsem).wait()
        pltpu.make_async_copy(ew_hbm.at[pl.ds(0, CR)], ew_st.at[p],
                              stsem).wait()

    stage(0, 0)

    def chunk(k, carry):
        p = lax.rem(k, 2)
        wait_st(p)
        @pl.when(k + 1 < nchunks)
        def _():
            stage(k + 1, 1 - p)
        for jj in range(CR):
            for g in range(8):
                src_st[p, jj, pl.ds(g * 16, 16)] = (
                    src_st[p, jj, pl.ds(g * 16, 16)] + row_off)
        @pl.when(k > 0)
        def _():
            pltpu.make_async_copy(rows_b, acc_sh.at[dst_st.at[p, 0]],
                                  ssem).wait()
        pltpu.async_copy(xw_hbm.at[src_st.at[p, 0]], rows_a, gsem)
        for jj in range(CR):
            cur = rows_a if jj % 2 == 0 else rows_b
            nxt = rows_b if jj % 2 == 0 else rows_a
            pltpu.make_async_copy(xw_hbm.at[src_st.at[p, jj]], cur,
                                  gsem).wait()
            if jj >= 1:
                pltpu.make_async_copy(nxt, acc_sh.at[dst_st.at[p, jj - 1]],
                                      ssem).wait()
            if jj + 1 < CR:
                pltpu.async_copy(xw_hbm.at[src_st.at[p, jj + 1]], nxt, gsem)
            _scale_rows(cur, ew_st, p, jj)
            pltpu.async_copy(cur, acc_sh.at[dst_st.at[p, jj]], ssem,
                             add=True)
        return carry

    lax.fori_loop(0, nchunks, chunk, 0)
    pltpu.make_async_copy(rows_b, acc_sh.at[dst_st.at[0, 0]], ssem).wait()
    plsc.subcore_barrier()
    @pl.when(s == 0)
    def _():
        pltpu.sync_copy(acc_sh, out_hbm.at[pl.ds(row_off, N)])


def _sc_main(xw_cat, src2d, dst2d, ew2d, zero2):
    f = pl.kernel(
        _main_body,
        out_type=jax.ShapeDtypeStruct((NCORE * N, CHUNK), jnp.float32),
        mesh=plsc.VectorSubcoreMesh(core_axis_name="c", subcore_axis_name="s"),
        scratch_types=[
            pltpu.VMEM((2, CR, 128), jnp.int32),
            pltpu.VMEM((2, CR, 128), jnp.int32),
            pltpu.VMEM((2, CR, 128), jnp.float32),
            pltpu.VMEM((128, CHUNK), jnp.float32),
            pltpu.VMEM((128, CHUNK), jnp.float32),
            pltpu.VMEM_SHARED((N, CHUNK), jnp.float32),
            pltpu.SemaphoreType.DMA,
            pltpu.SemaphoreType.DMA,
            pltpu.SemaphoreType.DMA,
        ],
    )
    return f(xw_cat, src2d, dst2d, ew2d, zero2)


def _front_body(xT_ref, w1_ref, b1_ref, wp_ref, deg_ref, out_ref, dinv_ref):
    nb = xT_ref.shape[1]
    xb = xT_ref[:].astype(jnp.bfloat16)
    wb = w1_ref[:].astype(jnp.bfloat16)
    y = jnp.dot(xb[0:13].reshape(13 * nb, 128), wb[0],
                preferred_element_type=jnp.float32)
    for k in range(1, 8):
        y = y + jnp.dot(xb[k:k + 13].reshape(13 * nb, 128), wb[k],
                        preferred_element_type=jnp.float32)
    y = (y + b1_ref[:]).reshape(13, nb, 128)
    cols = []
    for t in range(13):
        h = y[t][:, :64] * jax.nn.sigmoid(y[t][:, 64:])
        cols.append(jnp.dot(h, wp_ref[:], preferred_element_type=jnp.float32))
    d = deg_ref[:, 0] + deg_ref[:, 1] + 1.0
    dv = jnp.where(d > 0, lax.rsqrt(d), 0.0)[:, None]
    dinv_ref[:] = dv
    flat = jnp.concatenate(cols, axis=1) * dv
    pad = jnp.zeros((nb, CHUNK - REAL), jnp.float32)
    out_ref[0] = jnp.concatenate([flat[:, :REAL], pad], axis=1)
    out_ref[1] = jnp.concatenate([flat[:, REAL:], pad], axis=1)


def _tc_front(xT, w1k, b1r, wp, degT, nb=1000):
    grid = (N // nb,)
    return pl.pallas_call(
        _front_body,
        grid=grid,
        in_specs=[
            pl.BlockSpec((20, nb, 128), lambda i: (0, i, 0)),
            pl.BlockSpec((8, 128, 128), lambda i: (0, 0, 0)),
            pl.BlockSpec((1, 128), lambda i: (0, 0)),
            pl.BlockSpec((64, 16), lambda i: (0, 0)),
            pl.BlockSpec((nb, 2), lambda i: (i, 0)),
        ],
        out_specs=[pl.BlockSpec((2, nb, CHUNK), lambda i: (0, i, 0)),
                   pl.BlockSpec((nb, 1), lambda i: (i, 0))],
        out_shape=[jax.ShapeDtypeStruct((2, N, CHUNK), jnp.float32),
                   jax.ShapeDtypeStruct((N, 1), jnp.float32)],
    )(xT, w1k, b1r, wp, degT)


def _epi_body(acc_ref, xw_ref, dinv_ref, bsgc_ref, w2_ref, b2_ref, out_ref):
    a = jnp.concatenate([acc_ref[0][:, :REAL], acc_ref[1][:, :REAL]], axis=1)
    xwf = jnp.concatenate([xw_ref[0][:, :REAL], xw_ref[1][:, :REAL]], axis=1)
    dv = dinv_ref[:]
    z = dv * (a + xwf) + bsgc_ref[:]
    z = jnp.maximum(z, 0.0)
    for u in range(6):
        y = jnp.dot(z[:, u * 16:u * 16 + 128], w2_ref[:],
                    preferred_element_type=jnp.float32) + b2_ref[:]
        out_ref[u] = y[:, :64] * jax.nn.sigmoid(y[:, 64:])


def _tc_epi(acc, xw, dinv_col, bsgc, w2f, b2r, nb=1000):
    grid = (N // nb,)
    return pl.pallas_call(
        _epi_body,
        grid=grid,
        in_specs=[
            pl.BlockSpec((2, nb, CHUNK), lambda i: (0, i, 0)),
            pl.BlockSpec((2, nb, CHUNK), lambda i: (0, i, 0)),
            pl.BlockSpec((nb, 1), lambda i: (i, 0)),
            pl.BlockSpec((1, 208), lambda i: (0, 0)),
            pl.BlockSpec((128, 128), lambda i: (0, 0)),
            pl.BlockSpec((1, 128), lambda i: (0, 0)),
        ],
        out_specs=pl.BlockSpec((6, nb, 64), lambda i: (0, i, 0)),
        out_shape=jax.ShapeDtypeStruct((6, N, 64), jnp.float32),
    )(acc, xw, dinv_col, bsgc, w2f, b2r)


def kernel(x, edge_index, edge_attr, batch,
           w_tgc1, b_tgc1, w_sgc, b_sgc, w_tgc2, b_tgc2):
    src = edge_index[0]
    dst = edge_index[1]

    xT = jnp.transpose(x, (2, 0, 1))
    w1k = jnp.transpose(w_tgc1, (2, 1, 0))
    b1r = b_tgc1.reshape(1, 128)
    w2f = jnp.transpose(w_tgc2, (2, 1, 0)).reshape(128, 128)
    b2r = b_tgc2.reshape(1, 128)
    bsgc = jnp.tile(b_sgc, 13).reshape(1, 208)
    zero1 = jnp.zeros((N,), jnp.float32)
    zero2 = jnp.zeros((N, CHUNK), jnp.float32)

    dst2d = dst.reshape(ROWS, 128)
    ew2d = edge_attr.reshape(ROWS, 128)
    deg_p = _sc_deg(dst2d, ew2d, zero1)
    xw, dinv_col = _tc_front(xT, w1k, b1r, w_sgc, deg_p.T)
    xw_cat = xw.reshape(NCORE * N, CHUNK)

    acc_cat = _sc_main(xw_cat, src.reshape(ROWS, 128), dst2d, ew2d, zero2)
    acc = acc_cat.reshape(2, N, CHUNK)

    res = _tc_epi(acc, xw, dinv_col, bsgc, w2f, b2r)
    return jnp.transpose(res, (1, 2, 0))
